# baseline (speedup 1.0000x reference)
import jax
import jax.numpy as jnp
from jax.experimental import pallas as pl
from jax.experimental.pallas import tpu as pltpu

SEQ_LEN = 200
D_MODEL = 128
BATCH = 4096
BB = 32
NB = BATCH // BB
NSEM = 8


def _bcast_kernel(pos_ref, out_ref, scratch, sems):
    scratch[...] = jnp.broadcast_to(pos_ref[...][None], (BB, SEQ_LEN, D_MODEL))

    def _start(k, _):
        pltpu.make_async_copy(
            scratch, out_ref.at[pl.ds(k * BB, BB)], sems.at[k % NSEM]
        ).start()
        return _

    jax.lax.fori_loop(0, NB, _start, None)

    def _wait(k, _):
        pltpu.make_async_copy(
            scratch, out_ref.at[pl.ds(k * BB, BB)], sems.at[k % NSEM]
        ).wait()
        return _

    jax.lax.fori_loop(0, NB, _wait, None)


@jax.jit
def _run(pos_embed):
    return pl.pallas_call(
        _bcast_kernel,
        grid=(1,),
        in_specs=[
            pl.BlockSpec((SEQ_LEN, D_MODEL), lambda i: (0, 0)),
        ],
        out_specs=pl.BlockSpec(memory_space=pl.ANY),
        out_shape=jax.ShapeDtypeStruct((BATCH, SEQ_LEN, D_MODEL), jnp.float32),
        scratch_shapes=[
            pltpu.VMEM((BB, SEQ_LEN, D_MODEL), jnp.float32),
            pltpu.SemaphoreType.DMA((NSEM,)),
        ],
        compiler_params=pltpu.CompilerParams(
            dimension_semantics=("arbitrary",),
        ),
    )(pos_embed)


def kernel(batch_size, pos_embed, positions):
    return _run(pos_embed)
